# Initial kernel scaffold; baseline (speedup 1.0000x reference)
import jax
import jax.numpy as jnp
from jax.experimental import pallas as pl


def kernel(h_n, h_p, h_t, W_p, W_q, edge_index_interacts, edge_index_agg):
    raise NotImplementedError("write your pallas kernel here")


# speedup vs baseline: 2.7708x; 2.7708x over previous
import functools

import jax
import jax.numpy as jnp
import numpy as np
from jax import lax
from jax.experimental import pallas as pl
from jax.experimental.pallas import tpu as pltpu
from jax.experimental.pallas import tpu_sc as plsc

N = 10000
T = 10000
E1 = 320000
E2 = 320000
DIM = 128
ALPHA = 0.2

NC = 2
NS = 16
NW = NC * NS
L = 16

C1 = 80
C2 = 80
EPT1 = E1 // NS
EPW2 = E2 // NW
RPT = 632
CH = 40

_mesh = plsc.VectorSubcoreMesh(core_axis_name="c", subcore_axis_name="s")


_LANE_PERMS = np.stack(
    [np.arange(L, dtype=np.int32) ^ sh for sh in (8, 4, 2, 1)] * 2)


def _lanesum(v, perms):
    for p in perms:
        v = v + jnp.take_along_axis(v, p, axis=0)
    return v


@functools.partial(
    pl.kernel,
    out_type=jax.ShapeDtypeStruct((NC, N, DIM), jnp.float32),
    mesh=_mesh,
    scratch_types=[
        pltpu.VMEM((C1,), jnp.int32),
        pltpu.VMEM((C1,), jnp.int32),
        pltpu.VMEM((C1, DIM), jnp.float32),
        pltpu.VMEM((C1, DIM), jnp.float32),
        pltpu.VMEM((C1, DIM), jnp.float32),
        pltpu.VMEM((DIM,), jnp.float32),
        pltpu.VMEM((8, L), jnp.int32),
        pltpu.VMEM_SHARED((N, DIM), jnp.float32),
        pltpu.SemaphoreType.DMA,
        pltpu.SemaphoreType.DMA,
    ],
)
def _phase1(hn_hbm, src_hbm, dst_hbm, wp_hbm, lidx_hbm,
            o_hbm,
            srcv, dstv, hs, hd, sb, wbuf, lbuf, acc, sem1, sem2):
    cid = lax.axis_index("c")
    sid = lax.axis_index("s")

    zv = jnp.zeros((L,), jnp.float32)

    def zrow(i, _):
        for k in range(DIM // L):
            sb[i, pl.ds(k * L, L)] = zv
        return 0

    lax.fori_loop(0, C1, zrow, 0)
    r0 = jnp.minimum(sid * RPT, N - RPT)

    def zchunk(j, _):
        off = jnp.minimum(r0 + j * CH, N - CH)
        pltpu.sync_copy(sb.at[pl.ds(0, CH)], acc.at[pl.ds(off, CH)])
        return 0

    lax.fori_loop(0, RPT // CH + 1, zchunk, 0)
    pltpu.sync_copy(wp_hbm, wbuf)
    pltpu.sync_copy(lidx_hbm, lbuf)
    plsc.subcore_barrier()

    wv = [wbuf[pl.ds(k * L, L)] for k in range(DIM // L)]
    perms = [lbuf[k, pl.ds(0, L)] for k in range(4)]
    lane0 = jnp.bitwise_xor(perms[0], 8) == 0

    ebase = sid * EPT1

    def chunk(ci, _):
        base = ebase + ci * C1
        pltpu.sync_copy(src_hbm.at[pl.ds(base, C1)], srcv)
        pltpu.sync_copy(dst_hbm.at[pl.ds(base, C1)], dstv)
        pltpu.async_copy(hn_hbm.at[srcv], hs, sem1).wait()
        pltpu.async_copy(hn_hbm.at[dstv], hd, sem2).wait()

        def edge_body(e, _):
            av = wv[0] * hs[e, pl.ds(0, L)] * hd[e, pl.ds(0, L)]
            for k in range(1, DIM // L):
                av = av + wv[k] * hs[e, pl.ds(k * L, L)] * hd[e, pl.ds(k * L, L)]
            d = _lanesum(av, perms)
            d = jnp.where(d >= 0.0, d, ALPHA * d)
            ev = jnp.exp(d)

            @pl.when(cid == 0)
            def _():
                for k in range(DIM // L):
                    sb[e, pl.ds(k * L, L)] = hs[e, pl.ds(k * L, L)] * ev

            @pl.when(cid == 1)
            def _():
                sb[e, pl.ds(0, L)] = jnp.where(lane0, ev, 0.0)

            return 0

        lax.fori_loop(0, C1, edge_body, 0)

        pltpu.sync_copy(sb, acc.at[dstv], add=True)
        return 0

    lax.fori_loop(0, EPT1 // C1, chunk, 0)

    plsc.subcore_barrier()

    def ochunk(j, _):
        off = jnp.minimum(r0 + j * CH, N - CH)
        pltpu.sync_copy(acc.at[pl.ds(off, CH)], sb.at[pl.ds(0, CH)])
        pltpu.sync_copy(sb.at[pl.ds(0, CH)], o_hbm.at[cid, pl.ds(off, CH)])
        return 0

    lax.fori_loop(0, RPT // CH + 1, ochunk, 0)


def _combine_body(p_ref, wq_ref, gft_ref):
    num = p_ref[0]
    den = p_ref[1][:, 0:1]
    ft = num / (den + 1e-16)
    g = lax.dot_general(ft, wq_ref[:, :DIM],
                        dimension_numbers=(((1,), (1,)), ((), ())),
                        preferred_element_type=jnp.float32)
    gft_ref[:, :DIM] = g
    gft_ref[:, DIM:] = ft


def _combine(part, W_q):
    BN = 1000
    return pl.pallas_call(
        _combine_body,
        grid=(N // BN,),
        in_specs=[
            pl.BlockSpec((NC, BN, DIM), lambda i: (0, i, 0)),
            pl.BlockSpec((DIM, 2 * DIM), lambda i: (0, 0)),
        ],
        out_specs=pl.BlockSpec((BN, 2 * DIM), lambda i: (i, 0)),
        out_shape=jax.ShapeDtypeStruct((N, 2 * DIM), jnp.float32),
    )(part, W_q)


def _p2_body(hp_ref, wq_ref, p2_ref):
    p2_ref[...] = lax.dot_general(hp_ref[...], wq_ref[:, DIM:],
                                  dimension_numbers=(((1,), (1,)), ((), ())),
                                  preferred_element_type=jnp.float32)


def _p2(h_p, W_q):
    BE = 2000
    return pl.pallas_call(
        _p2_body,
        grid=(E2 // BE,),
        in_specs=[
            pl.BlockSpec((BE, DIM), lambda i: (i, 0)),
            pl.BlockSpec((DIM, 2 * DIM), lambda i: (0, 0)),
        ],
        out_specs=pl.BlockSpec((BE, DIM), lambda i: (i, 0)),
        out_shape=jax.ShapeDtypeStruct((E2, DIM), jnp.float32),
    )(h_p, W_q)


@functools.partial(
    pl.kernel,
    out_type=jax.ShapeDtypeStruct((NC, T, DIM), jnp.float32),
    mesh=_mesh,
    scratch_types=[
        pltpu.VMEM((C2,), jnp.int32),
        pltpu.VMEM((C2,), jnp.int32),
        pltpu.VMEM((C2, 2 * DIM), jnp.float32),
        pltpu.VMEM((C2, DIM), jnp.float32),
        pltpu.VMEM((C2, DIM), jnp.float32),
        pltpu.VMEM((8, L), jnp.int32),
        pltpu.VMEM_SHARED((T, DIM), jnp.float32),
        pltpu.SemaphoreType.DMA,
        pltpu.SemaphoreType.DMA,
    ],
)
def _phase2(gft_hbm, ht_hbm, p2_hbm, src_hbm, dst_hbm, lidx_hbm,
            out_hbm,
            srcv, dstv, gf, ht, p2v, lbuf, acc, sem1, sem2):
    cid = lax.axis_index("c")
    sid = lax.axis_index("s")
    wid = sid * NC + cid

    zv = jnp.zeros((L,), jnp.float32)

    def zrow(i, _):
        for k in range(DIM // L):
            p2v[i, pl.ds(k * L, L)] = zv
        return 0

    lax.fori_loop(0, C2, zrow, 0)
    r0 = jnp.minimum(sid * RPT, T - RPT)

    def zchunk(j, _):
        off = jnp.minimum(r0 + j * CH, T - CH)
        pltpu.sync_copy(p2v.at[pl.ds(0, CH)], acc.at[pl.ds(off, CH)])
        return 0

    lax.fori_loop(0, RPT // CH + 1, zchunk, 0)
    pltpu.sync_copy(lidx_hbm, lbuf)
    plsc.subcore_barrier()

    perms = [lbuf[k, pl.ds(0, L)] for k in range(4)]
    ebase = wid * EPW2

    def chunk(ci, _):
        base = ebase + ci * C2
        pltpu.sync_copy(src_hbm.at[pl.ds(base, C2)], srcv)
        pltpu.sync_copy(dst_hbm.at[pl.ds(base, C2)], dstv)
        pltpu.async_copy(gft_hbm.at[srcv], gf, sem1).wait()
        pltpu.async_copy(ht_hbm.at[dstv], ht, sem2).wait()
        pltpu.sync_copy(p2_hbm.at[pl.ds(base, C2)], p2v)

        def edge(e, _):
            av = jnp.zeros((L,), jnp.float32)
            for k in range(DIM // L):
                x = gf[e, pl.ds(k * L, L)] + p2v[e, pl.ds(k * L, L)]
                a = jnp.exp(-2.0 * jnp.abs(x))
                t = (1.0 - a) / (1.0 + a)
                t = jnp.where(x >= 0.0, t, -t)
                av = av + t * ht[e, pl.ds(k * L, L)]
            s = _lanesum(av, perms)
            for k in range(DIM // L):
                p2v[e, pl.ds(k * L, L)] = gf[e, pl.ds(DIM + k * L, L)] * s
            return 0

        lax.fori_loop(0, C2, edge, 0)
        pltpu.sync_copy(p2v, acc.at[dstv], add=True)
        return 0

    lax.fori_loop(0, EPW2 // C2, chunk, 0)

    plsc.subcore_barrier()

    def ochunk(j, _):
        off = jnp.minimum(r0 + j * CH, T - CH)
        pltpu.sync_copy(acc.at[pl.ds(off, CH)], p2v.at[pl.ds(0, CH)])
        pltpu.sync_copy(p2v.at[pl.ds(0, CH)], out_hbm.at[cid, pl.ds(off, CH)])
        return 0

    lax.fori_loop(0, RPT // CH + 1, ochunk, 0)


def _add_body(p_ref, o_ref):
    o_ref[...] = p_ref[0] + p_ref[1]


def _final_add(part):
    BT = 1000
    return pl.pallas_call(
        _add_body,
        grid=(T // BT,),
        in_specs=[pl.BlockSpec((NC, BT, DIM), lambda i: (0, i, 0))],
        out_specs=pl.BlockSpec((BT, DIM), lambda i: (i, 0)),
        out_shape=jax.ShapeDtypeStruct((T, DIM), jnp.float32),
    )(part)


def kernel(h_n, h_p, h_t, W_p, W_q, edge_index_interacts, edge_index_agg):
    src1 = edge_index_interacts[0]
    dst1 = edge_index_interacts[1]
    src2 = edge_index_agg[0]
    dst2 = edge_index_agg[1]
    wp = W_p.reshape(DIM)
    lidx = jnp.asarray(_LANE_PERMS)

    part1 = _phase1(h_n, src1, dst1, wp, lidx)
    gft = _combine(part1, W_q)
    p2 = _p2(h_p, W_q)
    part = _phase2(gft, h_t, p2, src2, dst2, lidx)
    return _final_add(part)


# speedup vs baseline: 3.0711x; 1.1083x over previous
import functools

import jax
import jax.numpy as jnp
import numpy as np
from jax import lax
from jax.experimental import pallas as pl
from jax.experimental.pallas import tpu as pltpu
from jax.experimental.pallas import tpu_sc as plsc

N = 10000
T = 10000
E1 = 320000
E2 = 320000
DIM = 128
ALPHA = 0.2

NC = 2
NS = 16
NW = NC * NS
L = 16

C1 = 80
C2 = 80
EPT1 = E1 // NS
EPW2 = E2 // NW
RPT = 632
CH = 40

_mesh = plsc.VectorSubcoreMesh(core_axis_name="c", subcore_axis_name="s")


_LANE_PERMS = np.stack(
    [np.arange(L, dtype=np.int32) ^ sh for sh in (8, 4, 2, 1)] * 2)


def _lanesum(v, perms):
    for p in perms:
        v = v + jnp.take_along_axis(v, p, axis=0)
    return v


@functools.partial(
    pl.kernel,
    out_type=jax.ShapeDtypeStruct((NC, N, DIM), jnp.float32),
    mesh=_mesh,
    scratch_types=[
        pltpu.VMEM((C1,), jnp.int32),
        pltpu.VMEM((C1,), jnp.int32),
        pltpu.VMEM((C1, DIM), jnp.float32),
        pltpu.VMEM((C1, DIM), jnp.float32),
        pltpu.VMEM((C1, DIM), jnp.float32),
        pltpu.VMEM((DIM,), jnp.float32),
        pltpu.VMEM((8, L), jnp.int32),
        pltpu.VMEM_SHARED((N, DIM), jnp.float32),
        pltpu.SemaphoreType.DMA,
        pltpu.SemaphoreType.DMA,
    ],
)
def _phase1(hn_hbm, src_hbm, dst_hbm, wp_hbm, lidx_hbm,
            o_hbm,
            srcv, dstv, hs, hd, sb, wbuf, lbuf, acc, sem1, sem2):
    cid = lax.axis_index("c")
    sid = lax.axis_index("s")

    zv = jnp.zeros((L,), jnp.float32)

    def zrow(i, _):
        for k in range(DIM // L):
            sb[i, pl.ds(k * L, L)] = zv
        return 0

    lax.fori_loop(0, C1, zrow, 0)
    r0 = jnp.minimum(sid * RPT, N - RPT)

    def zchunk(j, _):
        off = jnp.minimum(r0 + j * CH, N - CH)
        pltpu.sync_copy(sb.at[pl.ds(0, CH)], acc.at[pl.ds(off, CH)])
        return 0

    lax.fori_loop(0, RPT // CH + 1, zchunk, 0)
    pltpu.sync_copy(wp_hbm, wbuf)
    pltpu.sync_copy(lidx_hbm, lbuf)
    plsc.subcore_barrier()

    wv = [wbuf[pl.ds(k * L, L)] for k in range(DIM // L)]
    perms = [lbuf[k, pl.ds(0, L)] for k in range(4)]
    lane0 = jnp.bitwise_xor(perms[0], 8) == 0

    ebase = sid * EPT1

    def chunk(ci, _):
        base = ebase + ci * C1
        pltpu.sync_copy(src_hbm.at[pl.ds(base, C1)], srcv)
        pltpu.sync_copy(dst_hbm.at[pl.ds(base, C1)], dstv)
        cp1 = pltpu.async_copy(hn_hbm.at[srcv], hs, sem1)
        cp2 = pltpu.async_copy(hn_hbm.at[dstv], hd, sem2)
        cp1.wait()
        cp2.wait()

        def edge_body(e, _):
            av = wv[0] * hs[e, pl.ds(0, L)] * hd[e, pl.ds(0, L)]
            for k in range(1, DIM // L):
                av = av + wv[k] * hs[e, pl.ds(k * L, L)] * hd[e, pl.ds(k * L, L)]
            d = _lanesum(av, perms)
            d = jnp.where(d >= 0.0, d, ALPHA * d)
            ev = jnp.exp(d)

            @pl.when(cid == 0)
            def _():
                for k in range(DIM // L):
                    sb[e, pl.ds(k * L, L)] = hs[e, pl.ds(k * L, L)] * ev

            @pl.when(cid == 1)
            def _():
                sb[e, pl.ds(0, L)] = jnp.where(lane0, ev, 0.0)

            return 0

        lax.fori_loop(0, C1, edge_body, 0)

        pltpu.sync_copy(sb, acc.at[dstv], add=True)
        return 0

    lax.fori_loop(0, EPT1 // C1, chunk, 0)

    plsc.subcore_barrier()

    def ochunk(j, _):
        off = jnp.minimum(r0 + j * CH, N - CH)
        pltpu.sync_copy(acc.at[pl.ds(off, CH)], sb.at[pl.ds(0, CH)])
        pltpu.sync_copy(sb.at[pl.ds(0, CH)], o_hbm.at[cid, pl.ds(off, CH)])
        return 0

    lax.fori_loop(0, RPT // CH + 1, ochunk, 0)


def _combine_body(p_ref, wq_ref, gft_ref):
    num = p_ref[0]
    den = p_ref[1][:, 0:1]
    ft = num / (den + 1e-16)
    g = lax.dot_general(ft, wq_ref[:, :DIM],
                        dimension_numbers=(((1,), (1,)), ((), ())),
                        preferred_element_type=jnp.float32)
    gft_ref[:, :DIM] = g
    gft_ref[:, DIM:] = ft


def _combine(part, W_q):
    BN = 1000
    return pl.pallas_call(
        _combine_body,
        grid=(N // BN,),
        in_specs=[
            pl.BlockSpec((NC, BN, DIM), lambda i: (0, i, 0)),
            pl.BlockSpec((DIM, 2 * DIM), lambda i: (0, 0)),
        ],
        out_specs=pl.BlockSpec((BN, 2 * DIM), lambda i: (i, 0)),
        out_shape=jax.ShapeDtypeStruct((N, 2 * DIM), jnp.float32),
    )(part, W_q)


def _p2_body(hp_ref, wq_ref, p2_ref):
    p2_ref[...] = lax.dot_general(hp_ref[...], wq_ref[:, DIM:],
                                  dimension_numbers=(((1,), (1,)), ((), ())),
                                  preferred_element_type=jnp.float32)


def _p2(h_p, W_q):
    BE = 2000
    return pl.pallas_call(
        _p2_body,
        grid=(E2 // BE,),
        in_specs=[
            pl.BlockSpec((BE, DIM), lambda i: (i, 0)),
            pl.BlockSpec((DIM, 2 * DIM), lambda i: (0, 0)),
        ],
        out_specs=pl.BlockSpec((BE, DIM), lambda i: (i, 0)),
        out_shape=jax.ShapeDtypeStruct((E2, DIM), jnp.float32),
    )(h_p, W_q)


@functools.partial(
    pl.kernel,
    out_type=jax.ShapeDtypeStruct((NC, T, DIM), jnp.float32),
    mesh=_mesh,
    scratch_types=[
        pltpu.VMEM((C2,), jnp.int32),
        pltpu.VMEM((C2,), jnp.int32),
        pltpu.VMEM((C2, 2 * DIM), jnp.float32),
        pltpu.VMEM((C2, DIM), jnp.float32),
        pltpu.VMEM((C2, DIM), jnp.float32),
        pltpu.VMEM((8, L), jnp.int32),
        pltpu.VMEM_SHARED((T, DIM), jnp.float32),
        pltpu.SemaphoreType.DMA,
        pltpu.SemaphoreType.DMA,
    ],
)
def _phase2(gft_hbm, ht_hbm, p2_hbm, src_hbm, dst_hbm, lidx_hbm,
            out_hbm,
            srcv, dstv, gf, ht, p2v, lbuf, acc, sem1, sem2):
    cid = lax.axis_index("c")
    sid = lax.axis_index("s")
    wid = sid * NC + cid

    zv = jnp.zeros((L,), jnp.float32)

    def zrow(i, _):
        for k in range(DIM // L):
            p2v[i, pl.ds(k * L, L)] = zv
        return 0

    lax.fori_loop(0, C2, zrow, 0)
    r0 = jnp.minimum(sid * RPT, T - RPT)

    def zchunk(j, _):
        off = jnp.minimum(r0 + j * CH, T - CH)
        pltpu.sync_copy(p2v.at[pl.ds(0, CH)], acc.at[pl.ds(off, CH)])
        return 0

    lax.fori_loop(0, RPT // CH + 1, zchunk, 0)
    pltpu.sync_copy(lidx_hbm, lbuf)
    plsc.subcore_barrier()

    perms = [lbuf[k, pl.ds(0, L)] for k in range(4)]
    ebase = wid * EPW2

    def chunk(ci, _):
        base = ebase + ci * C2
        pltpu.sync_copy(src_hbm.at[pl.ds(base, C2)], srcv)
        pltpu.sync_copy(dst_hbm.at[pl.ds(base, C2)], dstv)
        cp1 = pltpu.async_copy(gft_hbm.at[srcv], gf, sem1)
        cp2 = pltpu.async_copy(ht_hbm.at[dstv], ht, sem2)
        pltpu.sync_copy(p2_hbm.at[pl.ds(base, C2)], p2v)
        cp1.wait()
        cp2.wait()

        def edge(e, _):
            av = jnp.zeros((L,), jnp.float32)
            for k in range(DIM // L):
                x = gf[e, pl.ds(k * L, L)] + p2v[e, pl.ds(k * L, L)]
                a = jnp.exp(-2.0 * jnp.abs(x))
                t = (1.0 - a) / (1.0 + a)
                t = jnp.where(x >= 0.0, t, -t)
                av = av + t * ht[e, pl.ds(k * L, L)]
            s = _lanesum(av, perms)
            for k in range(DIM // L):
                p2v[e, pl.ds(k * L, L)] = gf[e, pl.ds(DIM + k * L, L)] * s
            return 0

        lax.fori_loop(0, C2, edge, 0)
        pltpu.sync_copy(p2v, acc.at[dstv], add=True)
        return 0

    lax.fori_loop(0, EPW2 // C2, chunk, 0)

    plsc.subcore_barrier()

    def ochunk(j, _):
        off = jnp.minimum(r0 + j * CH, T - CH)
        pltpu.sync_copy(acc.at[pl.ds(off, CH)], p2v.at[pl.ds(0, CH)])
        pltpu.sync_copy(p2v.at[pl.ds(0, CH)], out_hbm.at[cid, pl.ds(off, CH)])
        return 0

    lax.fori_loop(0, RPT // CH + 1, ochunk, 0)


def _add_body(p_ref, o_ref):
    o_ref[...] = p_ref[0] + p_ref[1]


def _final_add(part):
    BT = 1000
    return pl.pallas_call(
        _add_body,
        grid=(T // BT,),
        in_specs=[pl.BlockSpec((NC, BT, DIM), lambda i: (0, i, 0))],
        out_specs=pl.BlockSpec((BT, DIM), lambda i: (i, 0)),
        out_shape=jax.ShapeDtypeStruct((T, DIM), jnp.float32),
    )(part)


def kernel(h_n, h_p, h_t, W_p, W_q, edge_index_interacts, edge_index_agg):
    src1 = edge_index_interacts[0]
    dst1 = edge_index_interacts[1]
    src2 = edge_index_agg[0]
    dst2 = edge_index_agg[1]
    wp = W_p.reshape(DIM)
    lidx = jnp.asarray(_LANE_PERMS)

    part1 = _phase1(h_n, src1, dst1, wp, lidx)
    gft = _combine(part1, W_q)
    p2 = _p2(h_p, W_q)
    part = _phase2(gft, h_t, p2, src2, dst2, lidx)
    return _final_add(part)


# speedup vs baseline: 3.2287x; 1.0513x over previous
import functools

import jax
import jax.numpy as jnp
import numpy as np
from jax import lax
from jax.experimental import pallas as pl
from jax.experimental.pallas import tpu as pltpu
from jax.experimental.pallas import tpu_sc as plsc

N = 10000
T = 10000
E1 = 320000
E2 = 320000
DIM = 128
ALPHA = 0.2

NC = 2
NS = 16
NW = NC * NS
L = 16

C1 = 80
C2 = 80
EPT1 = E1 // NS
EPW2 = E2 // NW
RPT = 632
CH = 40

_mesh = plsc.VectorSubcoreMesh(core_axis_name="c", subcore_axis_name="s")


_LANE_PERMS = np.stack(
    [np.arange(L, dtype=np.int32) ^ sh for sh in (8, 4, 2, 1)] * 2)


def _lanesum(v, perms):
    for p in perms:
        v = v + jnp.take_along_axis(v, p, axis=0)
    return v


@functools.partial(
    pl.kernel,
    out_type=jax.ShapeDtypeStruct((NC, N, DIM), jnp.float32),
    mesh=_mesh,
    scratch_types=[
        pltpu.VMEM((C1,), jnp.int32),
        pltpu.VMEM((C1,), jnp.int32),
        pltpu.VMEM((C1, DIM), jnp.float32),
        pltpu.VMEM((C1, DIM), jnp.float32),
        pltpu.VMEM((C1, DIM), jnp.float32),
        pltpu.VMEM((DIM,), jnp.float32),
        pltpu.VMEM((8, L), jnp.int32),
        pltpu.VMEM_SHARED((N, DIM), jnp.float32),
        pltpu.SemaphoreType.DMA,
        pltpu.SemaphoreType.DMA,
    ],
)
def _phase1(hn_hbm, src_hbm, dst_hbm, wp_hbm, lidx_hbm,
            o_hbm,
            srcv, dstv, hs, hd, sb, wbuf, lbuf, acc, sem1, sem2):
    cid = lax.axis_index("c")
    sid = lax.axis_index("s")

    zv = jnp.zeros((L,), jnp.float32)

    def zrow(i, _):
        for k in range(DIM // L):
            sb[i, pl.ds(k * L, L)] = zv
        return 0

    lax.fori_loop(0, C1, zrow, 0)
    r0 = jnp.minimum(sid * RPT, N - RPT)

    def zchunk(j, _):
        off = jnp.minimum(r0 + j * CH, N - CH)
        pltpu.sync_copy(sb.at[pl.ds(0, CH)], acc.at[pl.ds(off, CH)])
        return 0

    lax.fori_loop(0, RPT // CH + 1, zchunk, 0)
    pltpu.sync_copy(wp_hbm, wbuf)
    pltpu.sync_copy(lidx_hbm, lbuf)
    plsc.subcore_barrier()

    wv = [wbuf[pl.ds(k * L, L)] for k in range(DIM // L)]
    perms = [lbuf[k, pl.ds(0, L)] for k in range(4)]
    lane0 = jnp.bitwise_xor(perms[0], 8) == 0

    ebase = sid * EPT1

    def chunk(ci, _):
        base = ebase + ci * C1
        ci1 = pltpu.async_copy(src_hbm.at[pl.ds(base, C1)], srcv, sem1)
        ci2 = pltpu.async_copy(dst_hbm.at[pl.ds(base, C1)], dstv, sem2)
        ci1.wait()
        ci2.wait()
        cp1 = pltpu.async_copy(hn_hbm.at[srcv], hs, sem1)
        cp2 = pltpu.async_copy(hn_hbm.at[dstv], hd, sem2)
        cp1.wait()
        cp2.wait()

        def edge_body(e, _):
            av = wv[0] * hs[e, pl.ds(0, L)] * hd[e, pl.ds(0, L)]
            for k in range(1, DIM // L):
                av = av + wv[k] * hs[e, pl.ds(k * L, L)] * hd[e, pl.ds(k * L, L)]
            d = _lanesum(av, perms)
            d = jnp.where(d >= 0.0, d, ALPHA * d)
            ev = jnp.exp(d)

            @pl.when(cid == 0)
            def _():
                for k in range(DIM // L):
                    sb[e, pl.ds(k * L, L)] = hs[e, pl.ds(k * L, L)] * ev

            @pl.when(cid == 1)
            def _():
                sb[e, pl.ds(0, L)] = jnp.where(lane0, ev, 0.0)

            return 0

        lax.fori_loop(0, C1, edge_body, 0)

        pltpu.sync_copy(sb, acc.at[dstv], add=True)
        return 0

    lax.fori_loop(0, EPT1 // C1, chunk, 0)

    plsc.subcore_barrier()

    def ochunk(j, _):
        off = jnp.minimum(r0 + j * CH, N - CH)
        pltpu.sync_copy(acc.at[pl.ds(off, CH)], sb.at[pl.ds(0, CH)])
        pltpu.sync_copy(sb.at[pl.ds(0, CH)], o_hbm.at[cid, pl.ds(off, CH)])
        return 0

    lax.fori_loop(0, RPT // CH + 1, ochunk, 0)


def _combine_body(p_ref, wq_ref, gft_ref):
    num = p_ref[0]
    den = p_ref[1][:, 0:1]
    ft = num / (den + 1e-16)
    g = lax.dot_general(ft, wq_ref[:, :DIM],
                        dimension_numbers=(((1,), (1,)), ((), ())),
                        preferred_element_type=jnp.float32)
    gft_ref[:, :DIM] = g
    gft_ref[:, DIM:] = ft


def _combine(part, W_q):
    BN = 1000
    return pl.pallas_call(
        _combine_body,
        grid=(N // BN,),
        in_specs=[
            pl.BlockSpec((NC, BN, DIM), lambda i: (0, i, 0)),
            pl.BlockSpec((DIM, 2 * DIM), lambda i: (0, 0)),
        ],
        out_specs=pl.BlockSpec((BN, 2 * DIM), lambda i: (i, 0)),
        out_shape=jax.ShapeDtypeStruct((N, 2 * DIM), jnp.float32),
    )(part, W_q)


def _p2_body(hp_ref, wq_ref, p2_ref):
    p2_ref[...] = lax.dot_general(hp_ref[...], wq_ref[:, DIM:],
                                  dimension_numbers=(((1,), (1,)), ((), ())),
                                  preferred_element_type=jnp.float32)


def _p2(h_p, W_q):
    BE = 2000
    return pl.pallas_call(
        _p2_body,
        grid=(E2 // BE,),
        in_specs=[
            pl.BlockSpec((BE, DIM), lambda i: (i, 0)),
            pl.BlockSpec((DIM, 2 * DIM), lambda i: (0, 0)),
        ],
        out_specs=pl.BlockSpec((BE, DIM), lambda i: (i, 0)),
        out_shape=jax.ShapeDtypeStruct((E2, DIM), jnp.float32),
    )(h_p, W_q)


@functools.partial(
    pl.kernel,
    out_type=jax.ShapeDtypeStruct((NC, T, DIM), jnp.float32),
    mesh=_mesh,
    scratch_types=[
        pltpu.VMEM((C2,), jnp.int32),
        pltpu.VMEM((C2,), jnp.int32),
        pltpu.VMEM((C2, 2 * DIM), jnp.float32),
        pltpu.VMEM((C2, DIM), jnp.float32),
        pltpu.VMEM((C2, DIM), jnp.float32),
        pltpu.VMEM((8, L), jnp.int32),
        pltpu.VMEM_SHARED((T, DIM), jnp.float32),
        pltpu.SemaphoreType.DMA,
        pltpu.SemaphoreType.DMA,
    ],
)
def _phase2(gft_hbm, ht_hbm, p2_hbm, src_hbm, dst_hbm, lidx_hbm,
            out_hbm,
            srcv, dstv, gf, ht, p2v, lbuf, acc, sem1, sem2):
    cid = lax.axis_index("c")
    sid = lax.axis_index("s")
    wid = sid * NC + cid

    zv = jnp.zeros((L,), jnp.float32)

    def zrow(i, _):
        for k in range(DIM // L):
            p2v[i, pl.ds(k * L, L)] = zv
        return 0

    lax.fori_loop(0, C2, zrow, 0)
    r0 = jnp.minimum(sid * RPT, T - RPT)

    def zchunk(j, _):
        off = jnp.minimum(r0 + j * CH, T - CH)
        pltpu.sync_copy(p2v.at[pl.ds(0, CH)], acc.at[pl.ds(off, CH)])
        return 0

    lax.fori_loop(0, RPT // CH + 1, zchunk, 0)
    pltpu.sync_copy(lidx_hbm, lbuf)
    plsc.subcore_barrier()

    perms = [lbuf[k, pl.ds(0, L)] for k in range(4)]
    ebase = wid * EPW2

    def chunk(ci, _):
        base = ebase + ci * C2
        ci1 = pltpu.async_copy(src_hbm.at[pl.ds(base, C2)], srcv, sem1)
        ci2 = pltpu.async_copy(dst_hbm.at[pl.ds(base, C2)], dstv, sem2)
        ci1.wait()
        ci2.wait()
        cp1 = pltpu.async_copy(gft_hbm.at[srcv], gf, sem1)
        cp2 = pltpu.async_copy(ht_hbm.at[dstv], ht, sem2)
        pltpu.sync_copy(p2_hbm.at[pl.ds(base, C2)], p2v)
        cp1.wait()
        cp2.wait()

        def edge(e, _):
            av = jnp.zeros((L,), jnp.float32)
            for k in range(DIM // L):
                x = gf[e, pl.ds(k * L, L)] + p2v[e, pl.ds(k * L, L)]
                a = jnp.exp(-2.0 * jnp.abs(x))
                t = (1.0 - a) / (1.0 + a)
                t = jnp.where(x >= 0.0, t, -t)
                av = av + t * ht[e, pl.ds(k * L, L)]
            s = _lanesum(av, perms)
            for k in range(DIM // L):
                p2v[e, pl.ds(k * L, L)] = gf[e, pl.ds(DIM + k * L, L)] * s
            return 0

        lax.fori_loop(0, C2, edge, 0)
        pltpu.sync_copy(p2v, acc.at[dstv], add=True)
        return 0

    lax.fori_loop(0, EPW2 // C2, chunk, 0)

    plsc.subcore_barrier()

    def ochunk(j, _):
        off = jnp.minimum(r0 + j * CH, T - CH)
        pltpu.sync_copy(acc.at[pl.ds(off, CH)], p2v.at[pl.ds(0, CH)])
        pltpu.sync_copy(p2v.at[pl.ds(0, CH)], out_hbm.at[cid, pl.ds(off, CH)])
        return 0

    lax.fori_loop(0, RPT // CH + 1, ochunk, 0)


def _add_body(p_ref, o_ref):
    o_ref[...] = p_ref[0] + p_ref[1]


def _final_add(part):
    BT = 1000
    return pl.pallas_call(
        _add_body,
        grid=(T // BT,),
        in_specs=[pl.BlockSpec((NC, BT, DIM), lambda i: (0, i, 0))],
        out_specs=pl.BlockSpec((BT, DIM), lambda i: (i, 0)),
        out_shape=jax.ShapeDtypeStruct((T, DIM), jnp.float32),
    )(part)


def kernel(h_n, h_p, h_t, W_p, W_q, edge_index_interacts, edge_index_agg):
    src1 = edge_index_interacts[0]
    dst1 = edge_index_interacts[1]
    src2 = edge_index_agg[0]
    dst2 = edge_index_agg[1]
    wp = W_p.reshape(DIM)
    lidx = jnp.asarray(_LANE_PERMS)

    part1 = _phase1(h_n, src1, dst1, wp, lidx)
    gft = _combine(part1, W_q)
    p2 = _p2(h_p, W_q)
    part = _phase2(gft, h_t, p2, src2, dst2, lidx)
    return _final_add(part)
